# baseline (speedup 1.0000x reference)
import jax
import jax.numpy as jnp
from jax.experimental import pallas as pl
from jax.experimental.pallas import tpu as pltpu


def _cumprod_block(x_ref, o_ref):
    x = x_ref[...]
    lane = jax.lax.broadcasted_iota(jnp.int32, x.shape, 2)
    for s in (1, 2, 4, 8, 16, 32, 64):
        rolled = pltpu.roll(x, s, axis=2)
        x = x * jnp.where(lane >= s, rolled, 1.0)
    tot = jax.lax.broadcast_in_dim(x[:, :, 127], x.shape, (0, 1))
    sub = jax.lax.broadcasted_iota(jnp.int32, x.shape, 1)
    carry = jnp.where(sub >= 1, pltpu.roll(tot, 1, axis=1), 1.0)
    for s in (1, 2, 4):
        rolled = pltpu.roll(carry, s, axis=1)
        carry = carry * jnp.where(sub >= s, rolled, 1.0)
    o_ref[...] = x * carry


def kernel(x):
    m, n = x.shape
    block_rows = 256
    x3 = x.reshape(m, 8, n // 8)
    out = pl.pallas_call(
        _cumprod_block,
        out_shape=jax.ShapeDtypeStruct((m, 8, n // 8), x.dtype),
        grid=(m // block_rows,),
        in_specs=[pl.BlockSpec((block_rows, 8, n // 8), lambda i: (i, 0, 0))],
        out_specs=pl.BlockSpec((block_rows, 8, n // 8), lambda i: (i, 0, 0)),
    )(x3)
    return out.reshape(m, n)


# speedup vs baseline: 1.0854x; 1.0854x over previous
import jax
import jax.numpy as jnp
from jax.experimental import pallas as pl
from jax.experimental.pallas import tpu as pltpu

_CHUNK = 128


def _cumprod_block(x_ref, o_ref):
    rows, n = x_ref.shape
    lane = jax.lax.broadcasted_iota(jnp.int32, (rows, _CHUNK), 1)
    chunks = []
    for k in range(n // _CHUNK):
        xk = x_ref[:, k * _CHUNK:(k + 1) * _CHUNK]
        for s in (1, 2, 4, 8, 16, 32, 64):
            xk = xk * jnp.where(lane >= s, pltpu.roll(xk, s, axis=1), 1.0)
        chunks.append(xk)
    out = chunks[0]
    o_ref[:, 0:_CHUNK] = out
    for k in range(1, n // _CHUNK):
        carry = jax.lax.broadcast_in_dim(out[:, _CHUNK - 1], (rows, _CHUNK), (0,))
        out = chunks[k] * carry
        o_ref[:, k * _CHUNK:(k + 1) * _CHUNK] = out


def kernel(x):
    m, n = x.shape
    block_rows = 256
    return pl.pallas_call(
        _cumprod_block,
        out_shape=jax.ShapeDtypeStruct((m, n), x.dtype),
        grid=(m // block_rows,),
        in_specs=[pl.BlockSpec((block_rows, n), lambda i: (i, 0))],
        out_specs=pl.BlockSpec((block_rows, n), lambda i: (i, 0)),
    )(x)


# speedup vs baseline: 1.7270x; 1.5911x over previous
import jax
import jax.numpy as jnp
from jax.experimental import pallas as pl

_CHUNK = 128


def _cumprod_block(x_ref, o_ref):
    x = x_ref[...]
    rows, n = x.shape
    lane = jax.lax.broadcasted_iota(jnp.int32, x.shape, 1)
    lmod = jnp.bitwise_and(lane, _CHUNK - 1)
    for s in (1, 2, 4, 8, 16, 32, 64):
        shifted = jnp.concatenate(
            [jnp.ones((rows, s), x.dtype), x[:, :-s]], axis=1)
        x = x * jnp.where(lmod >= s, shifted, 1.0)
    out = x[:, :_CHUNK]
    o_ref[:, :_CHUNK] = out
    for k in range(1, n // _CHUNK):
        carry = jax.lax.broadcast_in_dim(
            out[:, _CHUNK - 1], (rows, _CHUNK), (0,))
        out = x[:, k * _CHUNK:(k + 1) * _CHUNK] * carry
        o_ref[:, k * _CHUNK:(k + 1) * _CHUNK] = out


def kernel(x):
    m, n = x.shape
    block_rows = 256
    return pl.pallas_call(
        _cumprod_block,
        out_shape=jax.ShapeDtypeStruct((m, n), x.dtype),
        grid=(m // block_rows,),
        in_specs=[pl.BlockSpec((block_rows, n), lambda i: (i, 0))],
        out_specs=pl.BlockSpec((block_rows, n), lambda i: (i, 0)),
    )(x)


# speedup vs baseline: 2.1576x; 1.2493x over previous
import jax
import jax.numpy as jnp
from jax.experimental import pallas as pl


def _cumprod_block(x_ref, o_ref):
    x = x_ref[...]
    n = x.shape[-1]
    s = 1
    while s < n:
        ones = jnp.ones(x.shape[:-1] + (s,), dtype=x.dtype)
        x = x * jnp.concatenate([ones, x[:, :-s]], axis=-1)
        s *= 2
    o_ref[...] = x


def kernel(x):
    m, n = x.shape
    block_rows = 512
    return pl.pallas_call(
        _cumprod_block,
        out_shape=jax.ShapeDtypeStruct((m, n), x.dtype),
        grid=(m // block_rows,),
        in_specs=[pl.BlockSpec((block_rows, n), lambda i: (i, 0))],
        out_specs=pl.BlockSpec((block_rows, n), lambda i: (i, 0)),
    )(x)
